# Initial kernel scaffold; baseline (speedup 1.0000x reference)
import jax
import jax.numpy as jnp
from jax.experimental import pallas as pl


def kernel(text_f, vis_f, edge_index, W_enc_t, b_enc_t, W_enc_v, b_enc_v, Wt0, bt0, Wt1, bt1, Wv0, bv0, Wv1, bv1, W_head, b_head):
    raise NotImplementedError("write your pallas kernel here")


# speedup vs baseline: 1.7967x; 1.7967x over previous
import functools

import jax
import jax.numpy as jnp
from jax import lax
from jax.experimental import pallas as pl
from jax.experimental.pallas import tpu as pltpu
from jax.experimental.pallas import tpu_sc as plsc

_N = 10000
_NPAD = 10240
_E = 320000
_D = 128
_K = 40
_NS = 16
_CH = (_E // _NS) // _K
_STRIPE = _NPAD // _NS
_ZR = 16
_DPK = _NPAD // 8
_BLK = 640
_GRID = _NPAD // _BLK


def _fill2d(buf, nrows, ncols, value):
    v = jnp.full((16,), value, jnp.float32)

    def body(r, carry):
        for c in range(ncols // 16):
            buf[r, pl.ds(c * 16, 16)] = v
        return carry

    lax.fori_loop(0, nrows, body, 0)


def _zero_stripe(zbuf, acc, sid):
    def body(i, carry):
        pltpu.sync_copy(zbuf, acc.at[pl.ds(sid * _STRIPE + i * _ZR, _ZR), :])
        return carry

    lax.fori_loop(0, _STRIPE // _ZR, body, 0)


def _copy_out(acc, out_hbm, sid):
    @pl.when(sid < _NS - 1)
    def _():
        pltpu.sync_copy(acc.at[pl.ds(sid * _STRIPE, _STRIPE), :],
                        out_hbm.at[pl.ds(sid * _STRIPE, _STRIPE), :])

    @pl.when(sid == _NS - 1)
    def _():
        r0 = (_NS - 1) * _STRIPE
        nr = _N - r0
        pltpu.sync_copy(acc.at[pl.ds(r0, nr), :], out_hbm.at[pl.ds(r0, nr), :])


def _sweep(table_hbm, edges_hbm, srcv, dstv, rows, acc, sem, cid, sid):
    base = cid * _E + sid * (_E // _NS)
    dbase = 2 * _E + sid * (_E // _NS)

    def body(j, carry):
        e0 = j * _K
        pltpu.sync_copy(edges_hbm.at[pl.ds(pl.multiple_of(base + e0, 8), _K)],
                        srcv)
        pltpu.sync_copy(edges_hbm.at[pl.ds(pl.multiple_of(dbase + e0, 8), _K)],
                        dstv)
        pltpu.async_copy(table_hbm.at[srcv], rows, sem).wait()
        pltpu.sync_copy(rows, acc.at[dstv], add=True)
        return carry

    lax.fori_loop(0, _CH, body, 0)


def _deg_sweep(edges_hbm, dstv, onesb, acc, sid):
    dbase = 2 * _E + sid * (_E // _NS)

    def body(j, carry):
        e0 = j * _K
        pltpu.sync_copy(edges_hbm.at[pl.ds(pl.multiple_of(dbase + e0, 8), _K)],
                        dstv)
        pltpu.sync_copy(onesb, acc.at[dstv], add=True)
        return carry

    lax.fori_loop(0, _CH, body, 0)


def _pack_deg(acc, degw, ework, deg_hbm, cid, sid):
    def grp(g, carry):
        r0 = pl.multiple_of(sid * _STRIPE + g * 8, 8)
        pltpu.sync_copy(acc.at[pl.ds(r0, 8), :], ework)
        for p in range(8):
            degw[g, pl.ds(p * 16, 16)] = ework[p, pl.ds(0, 16)]
        return carry

    lax.fori_loop(0, _STRIPE // 8, grp, 0)

    @pl.when(cid == 0)
    def _():
        pltpu.sync_copy(
            degw,
            deg_hbm.at[pl.ds(pl.multiple_of(sid * (_STRIPE // 8), 8),
                             _STRIPE // 8), :])


def _elementwise(acc, degw, bbuf, ework, a_hbm, cid, sid):
    def grp(g, carry):
        r0 = pl.multiple_of(sid * _STRIPE + g * 8, 8)
        pltpu.sync_copy(acc.at[pl.ds(r0, 8), :], ework)
        for p in range(8):
            inv = 1.0 / jnp.maximum(degw[g, pl.ds(p * 16, 16)], 1.0)
            for c in range(_D // 16):
                x = ework[p, pl.ds(c * 16, 16)]
                b = bbuf[0, pl.ds(c * 16, 16)]
                ework[p, pl.ds(c * 16, 16)] = jnp.maximum(x * inv + b, 0.0)
        off = pl.multiple_of(cid * _NPAD + r0, 8)
        pltpu.sync_copy(ework, a_hbm.at[pl.ds(off, 8), :])
        return carry

    lax.fori_loop(0, _STRIPE // 8, grp, 0)


def _make_graph_kernel():
    mesh = plsc.VectorSubcoreMesh(core_axis_name="c", subcore_axis_name="s")

    @functools.partial(
        pl.kernel,
        mesh=mesh,
        out_type=[
            jax.ShapeDtypeStruct((2, _N, _D), jnp.float32),
            jax.ShapeDtypeStruct((_DPK, _D), jnp.float32),
            jax.ShapeDtypeStruct((2 * _NPAD, _D), jnp.float32),
        ],
        scratch_types=[
            pltpu.VMEM((_K,), jnp.int32),
            pltpu.VMEM((_K,), jnp.int32),
            pltpu.VMEM((_K, _D), jnp.float32),
            pltpu.VMEM((_K, _D), jnp.float32),
            pltpu.VMEM((_ZR, _D), jnp.float32),
            pltpu.VMEM((1, _D), jnp.float32),
            pltpu.VMEM((8, _D), jnp.float32),
            pltpu.VMEM((_STRIPE // 8, _D), jnp.float32),
            pltpu.VMEM_SHARED((_NPAD, _D), jnp.float32),
            pltpu.SemaphoreType.DMA,
        ],
    )
    def graph(gcat_hbm, edges_hbm, b2_hbm,
              q_hbm, deg_hbm, a_hbm,
              srcv, dstv, rows, onesb, zbuf, bbuf, ework, degw,
              acc, sem):
        cid = lax.axis_index("c")
        sid = lax.axis_index("s")

        _fill2d(zbuf, _ZR, _D, 0.0)
        _fill2d(onesb, _K, _D, 1.0)
        _zero_stripe(zbuf, acc, sid)
        pltpu.sync_copy(b2_hbm.at[cid], bbuf)
        plsc.subcore_barrier()

        _deg_sweep(edges_hbm, dstv, onesb, acc, sid)
        plsc.subcore_barrier()
        _pack_deg(acc, degw, ework, deg_hbm, cid, sid)
        _zero_stripe(zbuf, acc, sid)
        plsc.subcore_barrier()

        _sweep(gcat_hbm, edges_hbm, srcv, dstv, rows, acc, sem, cid, sid)
        plsc.subcore_barrier()

        _elementwise(acc, degw, bbuf, ework, a_hbm, cid, sid)
        _zero_stripe(zbuf, acc, sid)
        plsc.subcore_barrier()

        _sweep(a_hbm, edges_hbm, srcv, dstv, rows, acc, sem, cid, sid)
        plsc.subcore_barrier()

        _copy_out(acc, q_hbm.at[cid], sid)

    return graph


_GRAPH = _make_graph_kernel()


def _enc_body(t_ref, v_ref, wet, bet, wev, bev, wt0, wv0, g_ref):
    h_t = jnp.maximum(
        jnp.dot(t_ref[...], wet[...], preferred_element_type=jnp.float32)
        + bet[...], 0.0)
    g_ref[0] = jnp.dot(h_t, wt0[...], preferred_element_type=jnp.float32)
    h_v = jnp.maximum(
        jnp.dot(v_ref[...], wev[...], preferred_element_type=jnp.float32)
        + bev[...], 0.0)
    g_ref[1] = jnp.dot(h_v, wv0[...], preferred_element_type=jnp.float32)


def _fin_body(q_ref, deg_ref, bt1, bv1, wt1, wv1, wh, bh, out_ref):
    inv = 1.0 / jnp.maximum(deg_ref[:, 0:1], 1.0)
    wt1f = jnp.dot(wt1[...], wh[...], preferred_element_type=jnp.float32) * 0.5
    wv1f = jnp.dot(wv1[...], wh[...], preferred_element_type=jnp.float32) * 0.5
    s = (jnp.dot(q_ref[0], wt1f, preferred_element_type=jnp.float32)
         + jnp.dot(q_ref[1], wv1f, preferred_element_type=jnp.float32))
    bprime = jnp.dot((bt1[...] + bv1[...]) * 0.5, wh[...],
                     preferred_element_type=jnp.float32) + bh[...]
    out_ref[...] = s * inv + bprime


def _full_spec(shape):
    nd = len(shape)
    return pl.BlockSpec(shape, lambda i: (0,) * nd)


def kernel(text_f, vis_f, edge_index, W_enc_t, b_enc_t, W_enc_v, b_enc_v,
           Wt0, bt0, Wt1, bt1, Wv0, bv0, Wv1, bv1, W_head, b_head):
    f32 = jnp.float32
    bet = b_enc_t.reshape(1, _D)
    bev = b_enc_v.reshape(1, _D)
    bt1r = bt1.reshape(1, _D)
    bv1r = bv1.reshape(1, _D)
    bhr = b_head.reshape(1, _D)
    b2 = jnp.stack([bt0, bv0]).reshape(2, 1, _D)
    src, dst = edge_index[0], edge_index[1]
    edgesf = jnp.concatenate([src, src + _NPAD, dst])

    g = pl.pallas_call(
        _enc_body,
        grid=(_GRID,),
        in_specs=[pl.BlockSpec((_BLK, _D), lambda i: (i, 0)),
                  pl.BlockSpec((_BLK, _D), lambda i: (i, 0)),
                  _full_spec((_D, _D)), _full_spec((1, _D)),
                  _full_spec((_D, _D)), _full_spec((1, _D)),
                  _full_spec((_D, _D)), _full_spec((_D, _D))],
        out_specs=pl.BlockSpec((2, _BLK, _D), lambda i: (0, i, 0)),
        out_shape=jax.ShapeDtypeStruct((2, _NPAD, _D), f32),
    )(text_f, vis_f, W_enc_t, bet, W_enc_v, bev, Wt0, Wv0)
    gcat = g.reshape(2 * _NPAD, _D)

    q, deg, _a = _GRAPH(gcat, edgesf, b2)

    deg16 = jnp.broadcast_to(
        deg.reshape(_DPK, 8, 16)[:, :, 0].reshape(_NPAD)[:_N, None], (_N, 16))

    out = pl.pallas_call(
        _fin_body,
        grid=(10,),
        in_specs=[pl.BlockSpec((2, 1000, _D), lambda i: (0, i, 0)),
                  pl.BlockSpec((1000, 16), lambda i: (i, 0)),
                  _full_spec((1, _D)), _full_spec((1, _D)),
                  _full_spec((_D, _D)), _full_spec((_D, _D)),
                  _full_spec((_D, _D)), _full_spec((1, _D))],
        out_specs=pl.BlockSpec((1000, _D), lambda i: (i, 0)),
        out_shape=jax.ShapeDtypeStruct((_N, _D), f32),
    )(q, deg16, bt1r, bv1r, Wt1, Wv1, W_head, bhr)

    return out


# speedup vs baseline: 3.5523x; 1.9772x over previous
import functools

import jax
import jax.numpy as jnp
from jax import lax
from jax.experimental import pallas as pl
from jax.experimental.pallas import tpu as pltpu
from jax.experimental.pallas import tpu_sc as plsc

_N = 10000
_NPAD = 10240
_E = 320000
_D = 128
_K = 40
_NS = 16
_CH = (_E // _NS) // _K
_STRIPE = _NPAD // _NS
_ZR = 16
_DPK = _NPAD // 8
_BLK = 640
_GRID = _NPAD // _BLK


def _fill2d(buf, nrows, ncols, value):
    v = jnp.full((16,), value, jnp.float32)

    def body(r, carry):
        for c in range(ncols // 16):
            buf[r, pl.ds(c * 16, 16)] = v
        return carry

    lax.fori_loop(0, nrows, body, 0)


def _zero_stripe(zbuf, acc, sid):
    def body(i, carry):
        pltpu.sync_copy(zbuf, acc.at[pl.ds(sid * _STRIPE + i * _ZR, _ZR), :])
        return carry

    lax.fori_loop(0, _STRIPE // _ZR, body, 0)


def _copy_out(acc, out_hbm, sid):
    @pl.when(sid < _NS - 1)
    def _():
        pltpu.sync_copy(acc.at[pl.ds(sid * _STRIPE, _STRIPE), :],
                        out_hbm.at[pl.ds(sid * _STRIPE, _STRIPE), :])

    @pl.when(sid == _NS - 1)
    def _():
        r0 = (_NS - 1) * _STRIPE
        nr = _N - r0
        pltpu.sync_copy(acc.at[pl.ds(r0, nr), :], out_hbm.at[pl.ds(r0, nr), :])


def _pipe_sweep(table_hbm, edges_hbm, sets, semI, acc, cid, sid, gather,
                ones_src=None):
    base = cid * _E + sid * (_E // _NS)
    dbase = 2 * _E + sid * (_E // _NS)

    def src_of(b):
        return sets[b][2] if gather else ones_src

    def idx_descs(j, b):
        e0 = j * _K
        ds_ = []
        if gather:
            ds_.append(pltpu.make_async_copy(
                edges_hbm.at[pl.ds(pl.multiple_of(base + e0, 8), _K)],
                sets[b][0], semI))
        ds_.append(pltpu.make_async_copy(
            edges_hbm.at[pl.ds(pl.multiple_of(dbase + e0, 8), _K)],
            sets[b][1], semI))
        return ds_

    def g_start(b):
        pltpu.async_copy(table_hbm.at[sets[b][0]], sets[b][2], sets[b][3])

    def g_wait(b):
        pltpu.make_async_copy(table_hbm.at[sets[b][0]], sets[b][2],
                              sets[b][3]).wait()

    def s_start(b):
        pltpu.async_copy(src_of(b), acc.at[sets[b][1]], sets[b][4], add=True)

    def s_wait(b):
        pltpu.make_async_copy(src_of(b), acc.at[sets[b][1]],
                              sets[b][4]).wait()

    def step(j, b, scat_wait, guard):
        nb = (b + 1) % 3
        if scat_wait:
            s_wait(nb)
        descs = idx_descs(j + 1, nb)
        if guard:
            @pl.when(j < _CH - 1)
            def _():
                for d in descs:
                    d.start()
        else:
            for d in descs:
                d.start()
        if gather:
            g_wait(b)
        s_start(b)
        if guard:
            @pl.when(j < _CH - 1)
            def _():
                for d in descs:
                    d.wait()
                if gather:
                    g_start(nb)
        else:
            for d in descs:
                d.wait()
            if gather:
                g_start(nb)

    e0d = idx_descs(0, 0)
    for d in e0d:
        d.start()
    for d in e0d:
        d.wait()
    if gather:
        g_start(0)
    step(0, 0, False, False)
    step(1, 1, False, False)

    def grp(i, carry):
        for bb in range(3):
            step(2 + i * 3 + bb, (2 + bb) % 3, True, True)
        return carry

    lax.fori_loop(0, (_CH - 2) // 3, grp, 0)
    s_wait((_CH - 2) % 3)
    s_wait((_CH - 1) % 3)


def _pack_deg(acc, degw, ework, deg_hbm, cid, sid):
    def grp(g, carry):
        r0 = pl.multiple_of(sid * _STRIPE + g * 8, 8)
        pltpu.sync_copy(acc.at[pl.ds(r0, 8), :], ework)
        for p in range(8):
            degw[g, pl.ds(p * 16, 16)] = ework[p, pl.ds(0, 16)]
        return carry

    lax.fori_loop(0, _STRIPE // 8, grp, 0)

    @pl.when(cid == 0)
    def _():
        pltpu.sync_copy(
            degw,
            deg_hbm.at[pl.ds(pl.multiple_of(sid * (_STRIPE // 8), 8),
                             _STRIPE // 8), :])


def _elementwise(acc, degw, bbuf, ework, a_hbm, cid, sid):
    def grp(g, carry):
        r0 = pl.multiple_of(sid * _STRIPE + g * 8, 8)
        pltpu.sync_copy(acc.at[pl.ds(r0, 8), :], ework)
        for p in range(8):
            inv = 1.0 / jnp.maximum(degw[g, pl.ds(p * 16, 16)], 1.0)
            for c in range(_D // 16):
                x = ework[p, pl.ds(c * 16, 16)]
                b = bbuf[0, pl.ds(c * 16, 16)]
                ework[p, pl.ds(c * 16, 16)] = jnp.maximum(x * inv + b, 0.0)
        off = pl.multiple_of(cid * _NPAD + r0, 8)
        pltpu.sync_copy(ework, a_hbm.at[pl.ds(off, 8), :])
        return carry

    lax.fori_loop(0, _STRIPE // 8, grp, 0)


def _make_graph_kernel():
    mesh = plsc.VectorSubcoreMesh(core_axis_name="c", subcore_axis_name="s")

    @functools.partial(
        pl.kernel,
        mesh=mesh,
        out_type=[
            jax.ShapeDtypeStruct((2, _N, _D), jnp.float32),
            jax.ShapeDtypeStruct((_DPK, _D), jnp.float32),
            jax.ShapeDtypeStruct((2 * _NPAD, _D), jnp.float32),
        ],
        scratch_types=[
            pltpu.VMEM((_K,), jnp.int32),
            pltpu.VMEM((_K,), jnp.int32),
            pltpu.VMEM((_K, _D), jnp.float32),
            pltpu.VMEM((_K,), jnp.int32),
            pltpu.VMEM((_K,), jnp.int32),
            pltpu.VMEM((_K, _D), jnp.float32),
            pltpu.VMEM((_K,), jnp.int32),
            pltpu.VMEM((_K,), jnp.int32),
            pltpu.VMEM((_K, _D), jnp.float32),
            pltpu.VMEM((_K, _D), jnp.float32),
            pltpu.VMEM((_ZR, _D), jnp.float32),
            pltpu.VMEM((1, _D), jnp.float32),
            pltpu.VMEM((8, _D), jnp.float32),
            pltpu.VMEM((_STRIPE // 8, _D), jnp.float32),
            pltpu.VMEM_SHARED((_NPAD, _D), jnp.float32),
            pltpu.SemaphoreType.DMA,
            pltpu.SemaphoreType.DMA,
            pltpu.SemaphoreType.DMA,
            pltpu.SemaphoreType.DMA,
            pltpu.SemaphoreType.DMA,
            pltpu.SemaphoreType.DMA,
            pltpu.SemaphoreType.DMA,
        ],
    )
    def graph(gcat_hbm, edges_hbm, b2_hbm,
              q_hbm, deg_hbm, a_hbm,
              s0s, s0d, s0r, s1s, s1d, s1r, s2s, s2d, s2r,
              onesb, zbuf, bbuf, ework, degw, acc,
              sg0, ss0, sg1, ss1, sg2, ss2, semI):
        cid = lax.axis_index("c")
        sid = lax.axis_index("s")
        sets = ((s0s, s0d, s0r, sg0, ss0),
                (s1s, s1d, s1r, sg1, ss1),
                (s2s, s2d, s2r, sg2, ss2))

        _fill2d(zbuf, _ZR, _D, 0.0)
        _fill2d(onesb, _K, _D, 1.0)
        _zero_stripe(zbuf, acc, sid)
        pltpu.sync_copy(b2_hbm.at[cid], bbuf)
        plsc.subcore_barrier()

        _pipe_sweep(gcat_hbm, edges_hbm, sets, semI, acc, cid, sid, False,
                    ones_src=onesb)
        plsc.subcore_barrier()
        _pack_deg(acc, degw, ework, deg_hbm, cid, sid)
        _zero_stripe(zbuf, acc, sid)
        plsc.subcore_barrier()

        _pipe_sweep(gcat_hbm, edges_hbm, sets, semI, acc, cid, sid, True)
        plsc.subcore_barrier()

        _elementwise(acc, degw, bbuf, ework, a_hbm, cid, sid)
        _zero_stripe(zbuf, acc, sid)
        plsc.subcore_barrier()

        _pipe_sweep(a_hbm, edges_hbm, sets, semI, acc, cid, sid, True)
        plsc.subcore_barrier()

        _copy_out(acc, q_hbm.at[cid], sid)

    return graph


_GRAPH = _make_graph_kernel()


def _enc_body(t_ref, v_ref, wet, bet, wev, bev, wt0, wv0, g_ref):
    h_t = jnp.maximum(
        jnp.dot(t_ref[...], wet[...], preferred_element_type=jnp.float32)
        + bet[...], 0.0)
    g_ref[0] = jnp.dot(h_t, wt0[...], preferred_element_type=jnp.float32)
    h_v = jnp.maximum(
        jnp.dot(v_ref[...], wev[...], preferred_element_type=jnp.float32)
        + bev[...], 0.0)
    g_ref[1] = jnp.dot(h_v, wv0[...], preferred_element_type=jnp.float32)


def _fin_body(q_ref, deg_ref, bt1, bv1, wt1, wv1, wh, bh, out_ref):
    inv = 1.0 / jnp.maximum(deg_ref[:, 0:1], 1.0)
    wt1f = jnp.dot(wt1[...], wh[...], preferred_element_type=jnp.float32) * 0.5
    wv1f = jnp.dot(wv1[...], wh[...], preferred_element_type=jnp.float32) * 0.5
    s = (jnp.dot(q_ref[0], wt1f, preferred_element_type=jnp.float32)
         + jnp.dot(q_ref[1], wv1f, preferred_element_type=jnp.float32))
    bprime = jnp.dot((bt1[...] + bv1[...]) * 0.5, wh[...],
                     preferred_element_type=jnp.float32) + bh[...]
    out_ref[...] = s * inv + bprime


def _full_spec(shape):
    nd = len(shape)
    return pl.BlockSpec(shape, lambda i: (0,) * nd)


def kernel(text_f, vis_f, edge_index, W_enc_t, b_enc_t, W_enc_v, b_enc_v,
           Wt0, bt0, Wt1, bt1, Wv0, bv0, Wv1, bv1, W_head, b_head):
    f32 = jnp.float32
    bet = b_enc_t.reshape(1, _D)
    bev = b_enc_v.reshape(1, _D)
    bt1r = bt1.reshape(1, _D)
    bv1r = bv1.reshape(1, _D)
    bhr = b_head.reshape(1, _D)
    b2 = jnp.stack([bt0, bv0]).reshape(2, 1, _D)
    src, dst = edge_index[0], edge_index[1]
    edgesf = jnp.concatenate([src, src + _NPAD, dst])

    g = pl.pallas_call(
        _enc_body,
        grid=(_GRID,),
        in_specs=[pl.BlockSpec((_BLK, _D), lambda i: (i, 0)),
                  pl.BlockSpec((_BLK, _D), lambda i: (i, 0)),
                  _full_spec((_D, _D)), _full_spec((1, _D)),
                  _full_spec((_D, _D)), _full_spec((1, _D)),
                  _full_spec((_D, _D)), _full_spec((_D, _D))],
        out_specs=pl.BlockSpec((2, _BLK, _D), lambda i: (0, i, 0)),
        out_shape=jax.ShapeDtypeStruct((2, _NPAD, _D), f32),
    )(text_f, vis_f, W_enc_t, bet, W_enc_v, bev, Wt0, Wv0)
    gcat = g.reshape(2 * _NPAD, _D)

    q, deg, _a = _GRAPH(gcat, edgesf, b2)

    deg16 = jnp.broadcast_to(
        deg.reshape(_DPK, 8, 16)[:, :, 0].reshape(_NPAD)[:_N, None], (_N, 16))

    out = pl.pallas_call(
        _fin_body,
        grid=(10,),
        in_specs=[pl.BlockSpec((2, 1000, _D), lambda i: (0, i, 0)),
                  pl.BlockSpec((1000, 16), lambda i: (i, 0)),
                  _full_spec((1, _D)), _full_spec((1, _D)),
                  _full_spec((_D, _D)), _full_spec((_D, _D)),
                  _full_spec((_D, _D)), _full_spec((1, _D))],
        out_specs=pl.BlockSpec((1000, _D), lambda i: (i, 0)),
        out_shape=jax.ShapeDtypeStruct((_N, _D), f32),
    )(q, deg16, bt1r, bv1r, Wt1, Wv1, W_head, bhr)

    return out


# speedup vs baseline: 5.3184x; 1.4972x over previous
import functools

import jax
import jax.numpy as jnp
from jax import lax
from jax.experimental import pallas as pl
from jax.experimental.pallas import tpu as pltpu
from jax.experimental.pallas import tpu_sc as plsc

_N = 10000
_NPAD = 10240
_E = 320000
_D = 128
_K = 80
_NS = 16
_CH = (_E // _NS) // _K
_STRIPE = _NPAD // _NS
_ZR = 16
_DPK = _NPAD // 8
_BLK = 640
_GRID = _NPAD // _BLK


def _fill2d(buf, nrows, ncols, value):
    v = jnp.full((16,), value, jnp.float32)

    def body(r, carry):
        for c in range(ncols // 16):
            buf[r, pl.ds(c * 16, 16)] = v
        return carry

    lax.fori_loop(0, nrows, body, 0)


def _zero_stripe(zbuf, acc, sid):
    def body(i, carry):
        pltpu.sync_copy(zbuf, acc.at[pl.ds(sid * _STRIPE + i * _ZR, _ZR), :])
        return carry

    lax.fori_loop(0, _STRIPE // _ZR, body, 0)


def _copy_out(acc, out_hbm, sid):
    @pl.when(sid < _NS - 1)
    def _():
        pltpu.sync_copy(acc.at[pl.ds(sid * _STRIPE, _STRIPE), :],
                        out_hbm.at[pl.ds(sid * _STRIPE, _STRIPE), :])

    @pl.when(sid == _NS - 1)
    def _():
        r0 = (_NS - 1) * _STRIPE
        nr = _N - r0
        pltpu.sync_copy(acc.at[pl.ds(r0, nr), :], out_hbm.at[pl.ds(r0, nr), :])


def _pipe_sweep(table_hbm, edges_hbm, sets, semI, acc, cid, sid, gather,
                ones_src=None):
    base = cid * _E + sid * (_E // _NS)
    dbase = 2 * _E + sid * (_E // _NS)

    def src_of(b):
        return sets[b][2] if gather else ones_src

    def idx_descs(j, b):
        e0 = j * _K
        ds_ = []
        if gather:
            ds_.append(pltpu.make_async_copy(
                edges_hbm.at[pl.ds(pl.multiple_of(base + e0, 8), _K)],
                sets[b][0], semI))
        ds_.append(pltpu.make_async_copy(
            edges_hbm.at[pl.ds(pl.multiple_of(dbase + e0, 8), _K)],
            sets[b][1], semI))
        return ds_

    def g_start(b):
        pltpu.async_copy(table_hbm.at[sets[b][0]], sets[b][2], sets[b][3])

    def g_wait(b):
        pltpu.make_async_copy(table_hbm.at[sets[b][0]], sets[b][2],
                              sets[b][3]).wait()

    def s_start(b):
        pltpu.async_copy(src_of(b), acc.at[sets[b][1]], sets[b][4], add=True)

    def s_wait(b):
        pltpu.make_async_copy(src_of(b), acc.at[sets[b][1]],
                              sets[b][4]).wait()

    def step(j, b, scat_wait, guard):
        nb = (b + 1) % 3
        if scat_wait:
            s_wait(nb)
        descs = idx_descs(j + 1, nb)
        if guard:
            @pl.when(j < _CH - 1)
            def _():
                for d in descs:
                    d.start()
        else:
            for d in descs:
                d.start()
        if gather:
            g_wait(b)
        s_start(b)
        if guard:
            @pl.when(j < _CH - 1)
            def _():
                for d in descs:
                    d.wait()
                if gather:
                    g_start(nb)
        else:
            for d in descs:
                d.wait()
            if gather:
                g_start(nb)

    e0d = idx_descs(0, 0)
    for d in e0d:
        d.start()
    for d in e0d:
        d.wait()
    if gather:
        g_start(0)
    step(0, 0, False, False)
    step(1, 1, False, False)

    def grp(i, carry):
        for bb in range(3):
            step(2 + i * 3 + bb, (2 + bb) % 3, True, True)
        return carry

    lax.fori_loop(0, (_CH - 2) // 3, grp, 0)
    rem = (_CH - 2) % 3
    for t in range(rem):
        j = _CH - rem + t
        step(j, j % 3, True, j == _CH - 1)
    s_wait((_CH - 2) % 3)
    s_wait((_CH - 1) % 3)


def _pack_deg(acc, degw, ework, deg_hbm, cid, sid):
    def grp(g, carry):
        r0 = pl.multiple_of(sid * _STRIPE + g * 8, 8)
        pltpu.sync_copy(acc.at[pl.ds(r0, 8), :], ework)
        for p in range(8):
            degw[g, pl.ds(p * 16, 16)] = ework[p, pl.ds(0, 16)]
        return carry

    lax.fori_loop(0, _STRIPE // 8, grp, 0)

    @pl.when(cid == 0)
    def _():
        pltpu.sync_copy(
            degw,
            deg_hbm.at[pl.ds(pl.multiple_of(sid * (_STRIPE // 8), 8),
                             _STRIPE // 8), :])


def _elementwise(acc, degw, bbuf, ework, a_hbm, cid, sid):
    def grp(g, carry):
        r0 = pl.multiple_of(sid * _STRIPE + g * 8, 8)
        pltpu.sync_copy(acc.at[pl.ds(r0, 8), :], ework)
        for p in range(8):
            inv = 1.0 / jnp.maximum(degw[g, pl.ds(p * 16, 16)], 1.0)
            for c in range(_D // 16):
                x = ework[p, pl.ds(c * 16, 16)]
                b = bbuf[0, pl.ds(c * 16, 16)]
                ework[p, pl.ds(c * 16, 16)] = jnp.maximum(x * inv + b, 0.0)
        off = pl.multiple_of(cid * _NPAD + r0, 8)
        pltpu.sync_copy(ework, a_hbm.at[pl.ds(off, 8), :])
        return carry

    lax.fori_loop(0, _STRIPE // 8, grp, 0)


def _make_graph_kernel():
    mesh = plsc.VectorSubcoreMesh(core_axis_name="c", subcore_axis_name="s")

    @functools.partial(
        pl.kernel,
        mesh=mesh,
        out_type=[
            jax.ShapeDtypeStruct((2, _N, _D), jnp.float32),
            jax.ShapeDtypeStruct((_DPK, _D), jnp.float32),
            jax.ShapeDtypeStruct((2 * _NPAD, _D), jnp.float32),
        ],
        scratch_types=[
            pltpu.VMEM((_K,), jnp.int32),
            pltpu.VMEM((_K,), jnp.int32),
            pltpu.VMEM((_K, _D), jnp.float32),
            pltpu.VMEM((_K,), jnp.int32),
            pltpu.VMEM((_K,), jnp.int32),
            pltpu.VMEM((_K, _D), jnp.float32),
            pltpu.VMEM((_K,), jnp.int32),
            pltpu.VMEM((_K,), jnp.int32),
            pltpu.VMEM((_K, _D), jnp.float32),
            pltpu.VMEM((_ZR, _D), jnp.float32),
            pltpu.VMEM((1, _D), jnp.float32),
            pltpu.VMEM((8, _D), jnp.float32),
            pltpu.VMEM((_STRIPE // 8, _D), jnp.float32),
            pltpu.VMEM_SHARED((_NPAD, _D), jnp.float32),
            pltpu.SemaphoreType.DMA,
            pltpu.SemaphoreType.DMA,
            pltpu.SemaphoreType.DMA,
            pltpu.SemaphoreType.DMA,
            pltpu.SemaphoreType.DMA,
            pltpu.SemaphoreType.DMA,
            pltpu.SemaphoreType.DMA,
        ],
    )
    def graph(gcat_hbm, edges_hbm, b2_hbm,
              q_hbm, deg_hbm, a_hbm,
              s0s, s0d, s0r, s1s, s1d, s1r, s2s, s2d, s2r,
              zbuf, bbuf, ework, degw, acc,
              sg0, ss0, sg1, ss1, sg2, ss2, semI):
        cid = lax.axis_index("c")
        sid = lax.axis_index("s")
        sets = ((s0s, s0d, s0r, sg0, ss0),
                (s1s, s1d, s1r, sg1, ss1),
                (s2s, s2d, s2r, sg2, ss2))

        _fill2d(zbuf, _ZR, _D, 0.0)
        _fill2d(s0r, _K, _D, 1.0)
        _zero_stripe(zbuf, acc, sid)
        pltpu.sync_copy(b2_hbm.at[cid], bbuf)
        plsc.subcore_barrier()

        _pipe_sweep(gcat_hbm, edges_hbm, sets, semI, acc, cid, sid, False,
                    ones_src=s0r)
        plsc.subcore_barrier()
        _pack_deg(acc, degw, ework, deg_hbm, cid, sid)
        _zero_stripe(zbuf, acc, sid)
        plsc.subcore_barrier()

        _pipe_sweep(gcat_hbm, edges_hbm, sets, semI, acc, cid, sid, True)
        plsc.subcore_barrier()

        _elementwise(acc, degw, bbuf, ework, a_hbm, cid, sid)
        _zero_stripe(zbuf, acc, sid)
        plsc.subcore_barrier()

        _pipe_sweep(a_hbm, edges_hbm, sets, semI, acc, cid, sid, True)
        plsc.subcore_barrier()

        _copy_out(acc, q_hbm.at[cid], sid)

    return graph


_GRAPH = _make_graph_kernel()


def _enc_body(t_ref, v_ref, wet, bet, wev, bev, wt0, wv0, g_ref):
    h_t = jnp.maximum(
        jnp.dot(t_ref[...], wet[...], preferred_element_type=jnp.float32)
        + bet[...], 0.0)
    g_ref[0] = jnp.dot(h_t, wt0[...], preferred_element_type=jnp.float32)
    h_v = jnp.maximum(
        jnp.dot(v_ref[...], wev[...], preferred_element_type=jnp.float32)
        + bev[...], 0.0)
    g_ref[1] = jnp.dot(h_v, wv0[...], preferred_element_type=jnp.float32)


def _fin_body(q_ref, deg_ref, bt1, bv1, wt1, wv1, wh, bh, out_ref):
    inv = 1.0 / jnp.maximum(deg_ref[:, 0:1], 1.0)
    wt1f = jnp.dot(wt1[...], wh[...], preferred_element_type=jnp.float32) * 0.5
    wv1f = jnp.dot(wv1[...], wh[...], preferred_element_type=jnp.float32) * 0.5
    s = (jnp.dot(q_ref[0], wt1f, preferred_element_type=jnp.float32)
         + jnp.dot(q_ref[1], wv1f, preferred_element_type=jnp.float32))
    bprime = jnp.dot((bt1[...] + bv1[...]) * 0.5, wh[...],
                     preferred_element_type=jnp.float32) + bh[...]
    out_ref[...] = s * inv + bprime


def _full_spec(shape):
    nd = len(shape)
    return pl.BlockSpec(shape, lambda i: (0,) * nd)


def kernel(text_f, vis_f, edge_index, W_enc_t, b_enc_t, W_enc_v, b_enc_v,
           Wt0, bt0, Wt1, bt1, Wv0, bv0, Wv1, bv1, W_head, b_head):
    f32 = jnp.float32
    bet = b_enc_t.reshape(1, _D)
    bev = b_enc_v.reshape(1, _D)
    bt1r = bt1.reshape(1, _D)
    bv1r = bv1.reshape(1, _D)
    bhr = b_head.reshape(1, _D)
    b2 = jnp.stack([bt0, bv0]).reshape(2, 1, _D)
    src, dst = edge_index[0], edge_index[1]
    edgesf = jnp.concatenate([src, src + _NPAD, dst])

    g = pl.pallas_call(
        _enc_body,
        grid=(_GRID,),
        in_specs=[pl.BlockSpec((_BLK, _D), lambda i: (i, 0)),
                  pl.BlockSpec((_BLK, _D), lambda i: (i, 0)),
                  _full_spec((_D, _D)), _full_spec((1, _D)),
                  _full_spec((_D, _D)), _full_spec((1, _D)),
                  _full_spec((_D, _D)), _full_spec((_D, _D))],
        out_specs=pl.BlockSpec((2, _BLK, _D), lambda i: (0, i, 0)),
        out_shape=jax.ShapeDtypeStruct((2, _NPAD, _D), f32),
    )(text_f, vis_f, W_enc_t, bet, W_enc_v, bev, Wt0, Wv0)
    gcat = g.reshape(2 * _NPAD, _D)

    q, deg, _a = _GRAPH(gcat, edgesf, b2)

    deg16 = jnp.broadcast_to(
        deg.reshape(_DPK, 8, 16)[:, :, 0].reshape(_NPAD)[:_N, None], (_N, 16))

    out = pl.pallas_call(
        _fin_body,
        grid=(10,),
        in_specs=[pl.BlockSpec((2, 1000, _D), lambda i: (0, i, 0)),
                  pl.BlockSpec((1000, 16), lambda i: (i, 0)),
                  _full_spec((1, _D)), _full_spec((1, _D)),
                  _full_spec((_D, _D)), _full_spec((_D, _D)),
                  _full_spec((_D, _D)), _full_spec((1, _D))],
        out_specs=pl.BlockSpec((1000, _D), lambda i: (i, 0)),
        out_shape=jax.ShapeDtypeStruct((_N, _D), f32),
    )(q, deg16, bt1r, bv1r, Wt1, Wv1, W_head, bhr)

    return out


# speedup vs baseline: 5.5532x; 1.0442x over previous
import functools

import jax
import jax.numpy as jnp
from jax import lax
from jax.experimental import pallas as pl
from jax.experimental.pallas import tpu as pltpu
from jax.experimental.pallas import tpu_sc as plsc

_N = 10000
_NPAD = 10240
_E = 320000
_D = 128
_K = 80
_NS = 16
_CH = (_E // _NS) // _K
_STRIPE = _NPAD // _NS
_ZR = 16
_DPK = _NPAD // 8
_BLK = 640
_GRID = _NPAD // _BLK


def _fill2d(buf, nrows, ncols, value):
    v = jnp.full((16,), value, jnp.float32)

    def body(r, carry):
        for c in range(ncols // 16):
            buf[r, pl.ds(c * 16, 16)] = v
        return carry

    lax.fori_loop(0, nrows, body, 0)


def _zero_stripe(zbuf, acc, sid, sem=None):
    if sem is None:
        def body(i, carry):
            pltpu.sync_copy(zbuf,
                            acc.at[pl.ds(sid * _STRIPE + i * _ZR, _ZR), :])
            return carry

        lax.fori_loop(0, _STRIPE // _ZR, body, 0)
        return

    def fire(i, carry):
        pltpu.async_copy(zbuf, acc.at[pl.ds(sid * _STRIPE + i * _ZR, _ZR), :],
                         sem)
        return carry

    lax.fori_loop(0, _STRIPE // _ZR, fire, 0)

    def drain(i, carry):
        pltpu.make_async_copy(
            zbuf, acc.at[pl.ds(sid * _STRIPE + i * _ZR, _ZR), :], sem).wait()
        return carry

    lax.fori_loop(0, _STRIPE // _ZR, drain, 0)


def _copy_out(acc, out_hbm, sid):
    @pl.when(sid < _NS - 1)
    def _():
        pltpu.sync_copy(acc.at[pl.ds(sid * _STRIPE, _STRIPE), :],
                        out_hbm.at[pl.ds(sid * _STRIPE, _STRIPE), :])

    @pl.when(sid == _NS - 1)
    def _():
        r0 = (_NS - 1) * _STRIPE
        nr = _N - r0
        pltpu.sync_copy(acc.at[pl.ds(r0, nr), :], out_hbm.at[pl.ds(r0, nr), :])


def _pipe_sweep(table_hbm, edges_hbm, sets, semI, acc, cid, sid, gather,
                ones_src=None):
    base = cid * _E + sid * (_E // _NS)
    dbase = 2 * _E + sid * (_E // _NS)

    def src_of(b):
        return sets[b][2] if gather else ones_src

    def idx_descs(j, b):
        e0 = j * _K
        ds_ = []
        if gather:
            ds_.append(pltpu.make_async_copy(
                edges_hbm.at[pl.ds(pl.multiple_of(base + e0, 8), _K)],
                sets[b][0], semI))
        ds_.append(pltpu.make_async_copy(
            edges_hbm.at[pl.ds(pl.multiple_of(dbase + e0, 8), _K)],
            sets[b][1], semI))
        return ds_

    def g_start(b):
        pltpu.async_copy(table_hbm.at[sets[b][0]], sets[b][2], sets[b][3])

    def g_wait(b):
        pltpu.make_async_copy(table_hbm.at[sets[b][0]], sets[b][2],
                              sets[b][3]).wait()

    def s_start(b):
        pltpu.async_copy(src_of(b), acc.at[sets[b][1]], sets[b][4], add=True)

    def s_wait(b):
        pltpu.make_async_copy(src_of(b), acc.at[sets[b][1]],
                              sets[b][4]).wait()

    def step(j, b, scat_wait, guard):
        nb = (b + 1) % 3
        if scat_wait:
            s_wait(nb)
        descs = idx_descs(j + 1, nb)
        if guard:
            @pl.when(j < _CH - 1)
            def _():
                for d in descs:
                    d.start()
        else:
            for d in descs:
                d.start()
        if gather:
            g_wait(b)
        s_start(b)
        if guard:
            @pl.when(j < _CH - 1)
            def _():
                for d in descs:
                    d.wait()
                if gather:
                    g_start(nb)
        else:
            for d in descs:
                d.wait()
            if gather:
                g_start(nb)

    e0d = idx_descs(0, 0)
    for d in e0d:
        d.start()
    for d in e0d:
        d.wait()
    if gather:
        g_start(0)
    step(0, 0, False, False)
    step(1, 1, False, False)

    def grp(i, carry):
        for bb in range(3):
            step(2 + i * 3 + bb, (2 + bb) % 3, True, True)
        return carry

    lax.fori_loop(0, (_CH - 2) // 3, grp, 0)
    rem = (_CH - 2) % 3
    for t in range(rem):
        j = _CH - rem + t
        step(j, j % 3, True, j == _CH - 1)
    s_wait((_CH - 2) % 3)
    s_wait((_CH - 1) % 3)


def _pack_deg(acc, degw, eworks, zbuf, sems, semZ, deg_hbm, cid, sid):
    def rd(g, b):
        r0 = pl.multiple_of(sid * _STRIPE + g * 8, 8)
        return pltpu.make_async_copy(acc.at[pl.ds(r0, 8), :], eworks[b],
                                     sems[b])

    def zr(g):
        r0 = pl.multiple_of(sid * _STRIPE + g * 8, 8)
        pltpu.async_copy(zbuf.at[pl.ds(0, 8), :], acc.at[pl.ds(r0, 8), :],
                         semZ)

    rd(0, 0).start()
    rd(1, 1).start()

    def grp(i, carry):
        for b in range(2):
            g = i * 2 + b
            rd(g, b).wait()
            for p in range(8):
                degw[g, pl.ds(p * 16, 16)] = eworks[b][p, pl.ds(0, 16)]
            zr(g)

            @pl.when(g + 2 < _STRIPE // 8)
            def _():
                rd(g + 2, b).start()
        return carry

    lax.fori_loop(0, _STRIPE // 16, grp, 0)

    def drz(g, carry):
        r0 = pl.multiple_of(sid * _STRIPE + g * 8, 8)
        pltpu.make_async_copy(zbuf.at[pl.ds(0, 8), :],
                              acc.at[pl.ds(r0, 8), :], semZ).wait()
        return carry

    lax.fori_loop(0, _STRIPE // 8, drz, 0)

    @pl.when(cid == 0)
    def _():
        pltpu.sync_copy(
            degw,
            deg_hbm.at[pl.ds(pl.multiple_of(sid * (_STRIPE // 8), 8),
                             _STRIPE // 8), :])


def _elementwise(acc, degw, bbuf, eworks, obufs, zbuf, sems, osems, semZ,
                 a_hbm, cid, sid):
    def rd(g, b):
        r0 = pl.multiple_of(sid * _STRIPE + g * 8, 8)
        return pltpu.make_async_copy(acc.at[pl.ds(r0, 8), :], eworks[b],
                                     sems[b])

    def wr(g, b):
        off = pl.multiple_of(cid * _NPAD + sid * _STRIPE + g * 8, 8)
        return pltpu.make_async_copy(obufs[b], a_hbm.at[pl.ds(off, 8), :],
                                     osems[b])

    def zr(g):
        r0 = pl.multiple_of(sid * _STRIPE + g * 8, 8)
        pltpu.async_copy(zbuf.at[pl.ds(0, 8), :], acc.at[pl.ds(r0, 8), :],
                         semZ)

    rd(0, 0).start()
    rd(1, 1).start()

    def grp(i, carry):
        for b in range(2):
            g = i * 2 + b
            rd(g, b).wait()

            @pl.when(g >= 2)
            def _():
                wr(g - 2, b).wait()

            for p in range(8):
                inv = 1.0 / jnp.maximum(degw[g, pl.ds(p * 16, 16)], 1.0)
                for c in range(_D // 16):
                    x = eworks[b][p, pl.ds(c * 16, 16)]
                    bv = bbuf[0, pl.ds(c * 16, 16)]
                    obufs[b][p, pl.ds(c * 16, 16)] = jnp.maximum(
                        x * inv + bv, 0.0)
            wr(g, b).start()
            zr(g)

            @pl.when(g + 2 < _STRIPE // 8)
            def _():
                rd(g + 2, b).start()
        return carry

    lax.fori_loop(0, _STRIPE // 16, grp, 0)
    wr(_STRIPE // 8 - 2, 0).wait()
    wr(_STRIPE // 8 - 1, 1).wait()

    def drz(g, carry):
        r0 = pl.multiple_of(sid * _STRIPE + g * 8, 8)
        pltpu.make_async_copy(zbuf.at[pl.ds(0, 8), :],
                              acc.at[pl.ds(r0, 8), :], semZ).wait()
        return carry

    lax.fori_loop(0, _STRIPE // 8, drz, 0)


def _make_graph_kernel():
    mesh = plsc.VectorSubcoreMesh(core_axis_name="c", subcore_axis_name="s")

    @functools.partial(
        pl.kernel,
        mesh=mesh,
        out_type=[
            jax.ShapeDtypeStruct((2, _N, _D), jnp.float32),
            jax.ShapeDtypeStruct((_DPK, _D), jnp.float32),
            jax.ShapeDtypeStruct((2 * _NPAD, _D), jnp.float32),
        ],
        scratch_types=[
            pltpu.VMEM((_K,), jnp.int32),
            pltpu.VMEM((_K,), jnp.int32),
            pltpu.VMEM((_K, _D), jnp.float32),
            pltpu.VMEM((_K,), jnp.int32),
            pltpu.VMEM((_K,), jnp.int32),
            pltpu.VMEM((_K, _D), jnp.float32),
            pltpu.VMEM((_K,), jnp.int32),
            pltpu.VMEM((_K,), jnp.int32),
            pltpu.VMEM((_K, _D), jnp.float32),
            pltpu.VMEM((_ZR, _D), jnp.float32),
            pltpu.VMEM((1, _D), jnp.float32),
            pltpu.VMEM((8, _D), jnp.float32),
            pltpu.VMEM((8, _D), jnp.float32),
            pltpu.VMEM((8, _D), jnp.float32),
            pltpu.VMEM((8, _D), jnp.float32),
            pltpu.VMEM((_STRIPE // 8, _D), jnp.float32),
            pltpu.VMEM_SHARED((_NPAD, _D), jnp.float32),
            pltpu.SemaphoreType.DMA,
            pltpu.SemaphoreType.DMA,
            pltpu.SemaphoreType.DMA,
            pltpu.SemaphoreType.DMA,
            pltpu.SemaphoreType.DMA,
            pltpu.SemaphoreType.DMA,
            pltpu.SemaphoreType.DMA,
        ],
    )
    def graph(gcat_hbm, edges_hbm, b2_hbm,
              q_hbm, deg_hbm, a_hbm,
              s0s, s0d, s0r, s1s, s1d, s1r, s2s, s2d, s2r,
              zbuf, bbuf, ew0, ew1, ob0, ob1, degw, acc,
              sg0, ss0, sg1, ss1, sg2, ss2, semI):
        cid = lax.axis_index("c")
        sid = lax.axis_index("s")
        sets = ((s0s, s0d, s0r, sg0, ss0),
                (s1s, s1d, s1r, sg1, ss1),
                (s2s, s2d, s2r, sg2, ss2))

        _fill2d(zbuf, _ZR, _D, 0.0)
        _fill2d(s0r, _K, _D, 1.0)
        _zero_stripe(zbuf, acc, sid, semI)
        pltpu.sync_copy(b2_hbm.at[cid], bbuf)
        plsc.subcore_barrier()

        _pipe_sweep(gcat_hbm, edges_hbm, sets, semI, acc, cid, sid, False,
                    ones_src=s0r)
        plsc.subcore_barrier()
        _pack_deg(acc, degw, (ew0, ew1), zbuf, (sg0, sg1), semI, deg_hbm,
                  cid, sid)
        plsc.subcore_barrier()

        _pipe_sweep(gcat_hbm, edges_hbm, sets, semI, acc, cid, sid, True)
        plsc.subcore_barrier()

        _elementwise(acc, degw, bbuf, (ew0, ew1), (ob0, ob1), zbuf,
                     (sg0, sg1), (ss0, ss1), semI, a_hbm, cid, sid)
        plsc.subcore_barrier()

        _pipe_sweep(a_hbm, edges_hbm, sets, semI, acc, cid, sid, True)
        plsc.subcore_barrier()

        _copy_out(acc, q_hbm.at[cid], sid)

    return graph


_GRAPH = _make_graph_kernel()


def _enc_body(t_ref, v_ref, wet, bet, wev, bev, wt0, wv0, g_ref):
    h_t = jnp.maximum(
        jnp.dot(t_ref[...], wet[...], preferred_element_type=jnp.float32)
        + bet[...], 0.0)
    g_ref[0] = jnp.dot(h_t, wt0[...], preferred_element_type=jnp.float32)
    h_v = jnp.maximum(
        jnp.dot(v_ref[...], wev[...], preferred_element_type=jnp.float32)
        + bev[...], 0.0)
    g_ref[1] = jnp.dot(h_v, wv0[...], preferred_element_type=jnp.float32)


def _fin_body(q_ref, deg_ref, bt1, bv1, wt1, wv1, wh, bh, out_ref):
    inv = 1.0 / jnp.maximum(deg_ref[:, 0:1], 1.0)
    wt1f = jnp.dot(wt1[...], wh[...], preferred_element_type=jnp.float32) * 0.5
    wv1f = jnp.dot(wv1[...], wh[...], preferred_element_type=jnp.float32) * 0.5
    s = (jnp.dot(q_ref[0], wt1f, preferred_element_type=jnp.float32)
         + jnp.dot(q_ref[1], wv1f, preferred_element_type=jnp.float32))
    bprime = jnp.dot((bt1[...] + bv1[...]) * 0.5, wh[...],
                     preferred_element_type=jnp.float32) + bh[...]
    out_ref[...] = s * inv + bprime


def _full_spec(shape):
    nd = len(shape)
    return pl.BlockSpec(shape, lambda i: (0,) * nd)


def kernel(text_f, vis_f, edge_index, W_enc_t, b_enc_t, W_enc_v, b_enc_v,
           Wt0, bt0, Wt1, bt1, Wv0, bv0, Wv1, bv1, W_head, b_head):
    f32 = jnp.float32
    bet = b_enc_t.reshape(1, _D)
    bev = b_enc_v.reshape(1, _D)
    bt1r = bt1.reshape(1, _D)
    bv1r = bv1.reshape(1, _D)
    bhr = b_head.reshape(1, _D)
    b2 = jnp.stack([bt0, bv0]).reshape(2, 1, _D)
    src, dst = edge_index[0], edge_index[1]
    edgesf = jnp.concatenate([src, src + _NPAD, dst])

    g = pl.pallas_call(
        _enc_body,
        grid=(_GRID,),
        in_specs=[pl.BlockSpec((_BLK, _D), lambda i: (i, 0)),
                  pl.BlockSpec((_BLK, _D), lambda i: (i, 0)),
                  _full_spec((_D, _D)), _full_spec((1, _D)),
                  _full_spec((_D, _D)), _full_spec((1, _D)),
                  _full_spec((_D, _D)), _full_spec((_D, _D))],
        out_specs=pl.BlockSpec((2, _BLK, _D), lambda i: (0, i, 0)),
        out_shape=jax.ShapeDtypeStruct((2, _NPAD, _D), f32),
    )(text_f, vis_f, W_enc_t, bet, W_enc_v, bev, Wt0, Wv0)
    gcat = g.reshape(2 * _NPAD, _D)

    q, deg, _a = _GRAPH(gcat, edgesf, b2)

    deg16 = jnp.broadcast_to(
        deg.reshape(_DPK, 8, 16)[:, :, 0].reshape(_NPAD)[:_N, None], (_N, 16))

    out = pl.pallas_call(
        _fin_body,
        grid=(10,),
        in_specs=[pl.BlockSpec((2, 1000, _D), lambda i: (0, i, 0)),
                  pl.BlockSpec((1000, 16), lambda i: (i, 0)),
                  _full_spec((1, _D)), _full_spec((1, _D)),
                  _full_spec((_D, _D)), _full_spec((_D, _D)),
                  _full_spec((_D, _D)), _full_spec((1, _D))],
        out_specs=pl.BlockSpec((1000, _D), lambda i: (i, 0)),
        out_shape=jax.ShapeDtypeStruct((_N, _D), f32),
    )(q, deg16, bt1r, bv1r, Wt1, Wv1, W_head, bhr)

    return out
